# baseline (speedup 1.0000x reference)
import functools

import jax
import jax.numpy as jnp
from jax import lax
from jax.experimental import pallas as pl
from jax.experimental.pallas import tpu as pltpu
from jax.experimental.pallas import tpu_sc as plsc

VOCAB = 1000000
EMBED = 32
BATCH = 4096
SEQ = 200

NC = 2
NS = 16
NW = NC * NS
PER_W = BATCH // NW
N_PAIRS = PER_W // 2


@functools.partial(
    pl.kernel,
    out_type=jax.ShapeDtypeStruct((BATCH, SEQ, EMBED), jnp.float32),
    mesh=plsc.VectorSubcoreMesh(
        core_axis_name="c", subcore_axis_name="s",
        num_cores=NC, num_subcores=NS),
    scratch_types=[
        pltpu.VMEM((SEQ,), jnp.int32),
        pltpu.VMEM((SEQ,), jnp.int32),
        pltpu.VMEM((SEQ, EMBED), jnp.float32),
        pltpu.VMEM((SEQ, EMBED), jnp.float32),
        pltpu.SemaphoreType.DMA,
        pltpu.SemaphoreType.DMA,
        pltpu.SemaphoreType.DMA,
        pltpu.SemaphoreType.DMA,
        pltpu.SemaphoreType.DMA,
        pltpu.SemaphoreType.DMA,
    ],
    compiler_params=pltpu.CompilerParams(use_tc_tiling_on_sc=False),
)
def _gather_kernel(table_hbm, idx_hbm, out_hbm,
                   i0, i1, r0, r1, si0, si1, sg0, sg1, so0, so1):
    idx_v = [i0, i1]
    rows_v = [r0, r1]
    isem = [si0, si1]
    gsem = [sg0, sg1]
    osem = [so0, so1]

    wid = lax.axis_index("s") * NC + lax.axis_index("c")
    base = wid * PER_W

    def idx_start(b, p):
        pltpu.async_copy(idx_hbm.at[base + b], idx_v[p], isem[p])

    idx_start(0, 0)

    @pl.loop(0, N_PAIRS)
    def _pair(jj):
        for p in range(2):
            b = jj * 2 + p
            row = base + b
            pltpu.make_async_copy(idx_hbm.at[row], idx_v[p], isem[p]).wait()

            @pl.when(jj > 0)
            def _():
                pltpu.make_async_copy(
                    rows_v[p], out_hbm.at[row - 2], osem[p]).wait()

            gather = pltpu.async_copy(table_hbm.at[idx_v[p]], rows_v[p],
                                      gsem[p])

            if p == 0:
                idx_start(b + 1, 1)
            else:
                @pl.when(jj < N_PAIRS - 1)
                def _():
                    idx_start(b + 1, 0)

            gather.wait()
            pltpu.async_copy(rows_v[p], out_hbm.at[row], osem[p])

    for p in range(2):
        b = base + PER_W - 2 + p
        pltpu.make_async_copy(rows_v[p], out_hbm.at[b], osem[p]).wait()


def kernel(indices, table):
    return _gather_kernel(table, indices)
